# baseline (speedup 1.0000x reference)
import functools

import jax
import jax.numpy as jnp
from jax import lax
from jax.experimental import pallas as pl
from jax.experimental.pallas import tpu as pltpu
from jax.experimental.pallas import tpu_sc as plsc

EMBED_DIM = 64
CHUNK = 128


@functools.partial(jax.jit, static_argnums=(2, 3, 4))
def _lookup(idx, table, n_workers, n_chunks, num_cores):
    D = table.shape[1]
    B = n_workers * n_chunks * CHUNK
    mesh = plsc.VectorSubcoreMesh(core_axis_name="c", subcore_axis_name="s")

    @functools.partial(
        pl.kernel,
        mesh=mesh,
        compiler_params=pltpu.CompilerParams(use_tc_tiling_on_sc=False),
        out_type=jax.ShapeDtypeStruct((B, D), jnp.float32),
        scratch_types=[
            pltpu.VMEM((n_chunks, CHUNK), jnp.int32),
            pltpu.VMEM((CHUNK, D), jnp.float32),
            pltpu.SemaphoreType.DMA,
        ],
    )
    def k(idx_hbm, table_hbm, out_hbm, idx_v, rows_v, gsem):
        wid = lax.axis_index("s") * num_cores + lax.axis_index("c")
        base = wid * (n_chunks * CHUNK)
        pltpu.sync_copy(idx_hbm.at[wid], idx_v)

        def body(j, carry):
            pltpu.async_copy(table_hbm.at[idx_v.at[j]], rows_v, gsem).wait()
            pltpu.sync_copy(rows_v, out_hbm.at[pl.ds(base + j * CHUNK, CHUNK)])
            return carry

        lax.fori_loop(0, n_chunks, body, 0)

    return k(idx, table)


def kernel(batch, word_embeddings):
    B0, B1 = batch.shape
    B = B0 * B1
    info = plsc.get_sparse_core_info()
    n_workers = info.num_cores * info.num_subcores
    n_chunks = B // (n_workers * CHUNK)
    assert n_workers * n_chunks * CHUNK == B
    idx = batch.astype(jnp.int32).reshape(n_workers, n_chunks, CHUNK)
    out = _lookup(idx, word_embeddings, n_workers, n_chunks, info.num_cores)
    return out.reshape(B0, B1, word_embeddings.shape[1])


# speedup vs baseline: 1.1161x; 1.1161x over previous
import functools

import jax
import jax.numpy as jnp
from jax import lax
from jax.experimental import pallas as pl
from jax.experimental.pallas import tpu as pltpu
from jax.experimental.pallas import tpu_sc as plsc

EMBED_DIM = 64
CHUNK = 128
DEPTH = 4
RING = 2 * DEPTH


@functools.partial(jax.jit, static_argnums=(2, 3, 4))
def _lookup(idx, table, n_workers, n_chunks, num_cores):
    D = table.shape[1]
    B = n_workers * n_chunks * CHUNK
    per_w = n_chunks * CHUNK
    mesh = plsc.VectorSubcoreMesh(core_axis_name="c", subcore_axis_name="s")
    n_groups = n_chunks // RING
    assert n_groups * RING == n_chunks and n_groups >= 2

    @functools.partial(
        pl.kernel,
        mesh=mesh,
        compiler_params=pltpu.CompilerParams(use_tc_tiling_on_sc=False),
        out_type=jax.ShapeDtypeStruct((B, D), jnp.float32),
        scratch_types=[
            pltpu.VMEM((n_chunks, CHUNK), jnp.int32),
            pltpu.VMEM((RING, CHUNK, D), jnp.float32),
            pltpu.SemaphoreType.DMA,
            pltpu.SemaphoreType.DMA,
        ],
    )
    def k(idx_hbm, table_hbm, out_hbm, idx_v, rows_v, gsem, wsem):
        wid = lax.axis_index("s") * num_cores + lax.axis_index("c")
        base = wid * per_w
        pltpu.sync_copy(idx_hbm.at[wid], idx_v)

        def g_start(j, b):
            pltpu.make_async_copy(
                table_hbm.at[idx_v.at[j]], rows_v.at[b], gsem).start()

        def g_wait(b):
            pltpu.make_async_copy(
                table_hbm.at[idx_v.at[0]], rows_v.at[b], gsem).wait()

        def w_start(j, b):
            pltpu.make_async_copy(
                rows_v.at[b], out_hbm.at[pl.ds(base + j * CHUNK, CHUNK)],
                wsem).start()

        def w_wait(b):
            pltpu.make_async_copy(
                rows_v.at[b], out_hbm.at[pl.ds(base, CHUNK)], wsem).wait()

        for b in range(DEPTH):
            g_start(b, b)

        for j in range(RING):
            g_wait(j % RING)
            if j >= DEPTH:
                w_wait((j - DEPTH) % RING)
            g_start(j + DEPTH, (j + DEPTH) % RING)
            w_start(j, j % RING)

        def group(g, carry):
            j0 = g * RING
            for b in range(RING):
                j = j0 + b
                g_wait(b)
                w_wait((b - DEPTH) % RING)
                g_start(j + DEPTH, (b + DEPTH) % RING)
                w_start(j, b)
            return carry

        lax.fori_loop(1, n_groups - 1, group, 0)

        j0 = (n_groups - 1) * RING
        for b in range(RING):
            j = j0 + b
            g_wait(b)
            w_wait((b - DEPTH) % RING)
            if b + DEPTH < RING:
                g_start(j + DEPTH, (b + DEPTH) % RING)
            w_start(j, b)
        for b in range(RING - DEPTH, RING):
            w_wait(b)

    return k(idx, table)


def kernel(batch, word_embeddings):
    B0, B1 = batch.shape
    B = B0 * B1
    info = plsc.get_sparse_core_info()
    n_workers = info.num_cores * info.num_subcores
    n_chunks = B // (n_workers * CHUNK)
    assert n_workers * n_chunks * CHUNK == B
    idx = batch.astype(jnp.int32).reshape(n_workers, n_chunks, CHUNK)
    out = _lookup(idx, word_embeddings, n_workers, n_chunks, info.num_cores)
    return out.reshape(B0, B1, word_embeddings.shape[1])
